# baseline (speedup 1.0000x reference)
import functools

import jax
import jax.numpy as jnp
from jax import lax
from jax.experimental import pallas as pl
from jax.experimental.pallas import tpu as pltpu
from jax.experimental.pallas import tpu_sc as plsc

EPS = 1e-09
N = 10000
E = 640000
NW = 32
CH = 128
EP = 655360
NACC = 10240
NBUF = 5
LOOK = 3
BE = 2560
BM = 1024
BN = 1280


def _sc_gather(table, idx):
    nch = idx.shape[1]
    ep = NW * nch * CH
    d = table.shape[1]
    mesh = plsc.VectorSubcoreMesh(core_axis_name="c", subcore_axis_name="s")

    def out_at(ref, j):
        if d == 128:
            return ref.at[pl.ds(j * CH, CH)]
        return ref.at[pl.ds(j * CH, CH), pl.ds(0, d)]

    @functools.partial(
        pl.kernel,
        out_type=jax.ShapeDtypeStruct((ep, 128), jnp.float32),
        mesh=mesh,
        compiler_params=pltpu.CompilerParams(use_tc_tiling_on_sc=False),
        scratch_types=[
            pltpu.VMEM((nch, CH), jnp.int32),
            pltpu.VMEM((NBUF, CH, d), jnp.float32),
            pltpu.SemaphoreType.DMA((NBUF,)),
            pltpu.SemaphoreType.DMA((NBUF,)),
        ],
    )
    def k(table_hbm, idx_hbm, out_hbm, idx_v, bufs_v, gsem, ssem):
        wid = lax.axis_index("s") * 2 + lax.axis_index("c")
        row0 = wid * nch
        pltpu.sync_copy(idx_hbm.at[wid], idx_v)

        for j in range(LOOK):
            pltpu.async_copy(table_hbm.at[idx_v.at[j]], bufs_v.at[j], gsem.at[j])

        def group(g, carry):
            for b in range(NBUF):
                j = g * NBUF + b
                pltpu.make_async_copy(
                    table_hbm.at[idx_v.at[0]], bufs_v.at[b], gsem.at[b]).wait()
                pltpu.async_copy(
                    bufs_v.at[b], out_at(out_hbm, row0 + j), ssem.at[b])
                jn = j + LOOK
                bn = (b + LOOK) % NBUF

                @pl.when(jn < nch)
                def _():
                    @pl.when(jn >= NBUF)
                    def _():
                        pltpu.make_async_copy(
                            bufs_v.at[bn], out_at(out_hbm, 0),
                            ssem.at[bn]).wait()
                    pltpu.async_copy(
                        table_hbm.at[idx_v.at[jn]], bufs_v.at[bn], gsem.at[bn])
            return carry

        lax.fori_loop(0, nch // NBUF, group, 0)
        for b in range(NBUF):
            pltpu.make_async_copy(
                bufs_v.at[b], out_at(out_hbm, 0), ssem.at[b]).wait()

    return k(table, idx)


def _sc_scatter_add(vals, idx, zeros):
    nch = idx.shape[1]
    d = zeros.shape[1]
    rpt = NACC // 16
    mesh = plsc.VectorSubcoreMesh(core_axis_name="c", subcore_axis_name="s")

    def vals_at(ref, j):
        return ref.at[pl.ds(j * CH, CH), pl.ds(0, d)]

    @functools.partial(
        pl.kernel,
        out_type=jax.ShapeDtypeStruct((2, NACC, d), jnp.float32),
        mesh=mesh,
        compiler_params=pltpu.CompilerParams(use_tc_tiling_on_sc=False),
        scratch_types=[
            pltpu.VMEM((nch, CH), jnp.int32),
            pltpu.VMEM((NBUF, CH, d), jnp.float32),
            pltpu.VMEM_SHARED((NACC, d), jnp.float32),
            pltpu.SemaphoreType.DMA((NBUF,)),
            pltpu.SemaphoreType.DMA((NBUF,)),
        ],
    )
    def k(vals_hbm, idx_hbm, zeros_hbm, out_hbm, idx_v, bufs_v, acc_sh, gsem, ssem):
        cid = lax.axis_index("c")
        sid = lax.axis_index("s")
        wid = sid * 2 + cid
        row0 = wid * nch
        pltpu.sync_copy(zeros_hbm.at[pl.ds(sid * rpt, rpt)],
                        acc_sh.at[pl.ds(sid * rpt, rpt)])
        plsc.subcore_barrier()
        pltpu.sync_copy(idx_hbm.at[wid], idx_v)

        for j in range(LOOK):
            pltpu.async_copy(
                vals_at(vals_hbm, row0 + j), bufs_v.at[j], gsem.at[j])

        def group(g, carry):
            for b in range(NBUF):
                j = g * NBUF + b
                pltpu.make_async_copy(
                    vals_at(vals_hbm, 0), bufs_v.at[b], gsem.at[b]).wait()
                pltpu.async_copy(
                    bufs_v.at[b], acc_sh.at[idx_v.at[j]], ssem.at[b], add=True)
                jn = j + LOOK
                bn = (b + LOOK) % NBUF

                @pl.when(jn < nch)
                def _():
                    @pl.when(jn >= NBUF)
                    def _():
                        pltpu.make_async_copy(
                            bufs_v.at[bn], acc_sh.at[idx_v.at[0]],
                            ssem.at[bn]).wait()
                    pltpu.async_copy(
                        vals_at(vals_hbm, row0 + jn), bufs_v.at[bn],
                        gsem.at[bn])
            return carry

        lax.fori_loop(0, nch // NBUF, group, 0)
        for b in range(NBUF):
            pltpu.make_async_copy(
                bufs_v.at[b], acc_sh.at[idx_v.at[0]], ssem.at[b]).wait()
        plsc.subcore_barrier()
        pltpu.sync_copy(acc_sh.at[pl.ds(sid * rpt, rpt)],
                        out_hbm.at[cid].at[pl.ds(sid * rpt, rpt)])

    return k(vals, idx, zeros)


def _make_edge(dg):
    def body(g_ref, ea_ref, w_ref, b_ref, nw_ref, o_ref):
        e = jnp.dot(ea_ref[...], w_ref[...], preferred_element_type=jnp.float32)
        m = jax.nn.relu(g_ref[:, :dg] + e + b_ref[...])
        o_ref[...] = jnp.dot(m, nw_ref[...], preferred_element_type=jnp.float32)

    def edge(g, ea, w, b, nw):
        return pl.pallas_call(
            body,
            grid=(EP // BE,),
            in_specs=[
                pl.BlockSpec((BE, 128), lambda i: (i, 0)),
                pl.BlockSpec((BE, 16), lambda i: (i, 0)),
                pl.BlockSpec((16, dg), lambda i: (0, 0)),
                pl.BlockSpec((1, dg), lambda i: (0, 0)),
                pl.BlockSpec((dg, 128), lambda i: (0, 0)),
            ],
            out_specs=pl.BlockSpec((BE, 128), lambda i: (i, 0)),
            out_shape=jax.ShapeDtypeStruct((EP, 128), jnp.float32),
        )(g, ea, w, b, nw)

    return edge


_edge1 = _make_edge(128)
_edge2 = _make_edge(64)


def _gram_body(a_ref, b_ref, o_ref):
    o_ref[...] = jax.lax.dot_general(
        a_ref[...], b_ref[...], (((1,), (1,)), ((), ())),
        preferred_element_type=jnp.float32)


def _gram(h):
    n = h.shape[0]
    grid = (pl.cdiv(n, BM), pl.cdiv(n, BN))
    return pl.pallas_call(
        _gram_body,
        grid=grid,
        in_specs=[
            pl.BlockSpec((BM, h.shape[1]), lambda i, j: (i, 0)),
            pl.BlockSpec((BN, h.shape[1]), lambda i, j: (j, 0)),
        ],
        out_specs=pl.BlockSpec((BM, BN), lambda i, j: (i, j)),
        out_shape=jax.ShapeDtypeStruct((n, n), jnp.float32),
    )(h, h)


def _pad128(w):
    return jnp.pad(w, ((0, 0), (0, 128 - w.shape[1])))


def kernel(x, edge_index, edge_attr, emb, lin_edge1_w, lin_edge1_b, nn1_w, nn1_b,
           lin_edge2_w, lin_edge2_b, nn2_w, nn2_b):
    src = jnp.concatenate([edge_index[0], jnp.zeros((EP - E,), jnp.int32)])
    dst = jnp.concatenate(
        [edge_index[1], jnp.full((EP - E,), N, jnp.int32)])
    nch = EP // (NW * CH)
    src3 = src.reshape(NW, nch, CH)
    dst3 = dst.reshape(NW, nch, CH)
    ea = jnp.pad(edge_attr, ((0, EP - E), (0, 0)))

    hx = jnp.take(emb, x[:, 0], axis=0)

    hs = _sc_gather(hx, src3)
    p1 = _edge1(hs, ea, lin_edge1_w, lin_edge1_b.reshape(1, 128),
                _pad128(nn1_w))
    part = _sc_scatter_add(p1, dst3, jnp.zeros((NACC, 64), jnp.float32))
    aggr = (part[0] + part[1])[:N]
    h1 = jax.nn.leaky_relu(aggr + (1.0 + EPS) * (hx @ nn1_w) + nn1_b,
                           negative_slope=0.01)

    g = _sc_gather(h1, src3)
    p2 = _edge2(g, ea, lin_edge2_w, lin_edge2_b.reshape(1, 64),
                _pad128(nn2_w))
    part = _sc_scatter_add(p2, dst3, jnp.zeros((NACC, 32), jnp.float32))
    aggr = (part[0] + part[1])[:N]
    h2 = aggr + (1.0 + EPS) * (h1 @ nn2_w) + nn2_b

    return _gram(h2)
